# Initial kernel scaffold; baseline (speedup 1.0000x reference)
import jax
import jax.numpy as jnp
from jax.experimental import pallas as pl


def kernel(x, table):
    raise NotImplementedError("write your pallas kernel here")


# speedup vs baseline: 1.6812x; 1.6812x over previous
import jax
import jax.numpy as jnp
from jax.experimental import pallas as pl

_TOKEN_BLOCK = 1024


def _add_kernel(x_ref, t_ref, o_ref):
    o_ref[0] = x_ref[0] + t_ref[...]


def kernel(x, table):
    batch, num_tokens, token_size = x.shape
    tb = _TOKEN_BLOCK
    grid = (num_tokens // tb, batch)
    return pl.pallas_call(
        _add_kernel,
        grid=grid,
        in_specs=[
            pl.BlockSpec((1, tb, token_size), lambda i, j: (j, i, 0)),
            pl.BlockSpec((tb, token_size), lambda i, j: (i, 0)),
        ],
        out_specs=pl.BlockSpec((1, tb, token_size), lambda i, j: (j, i, 0)),
        out_shape=jax.ShapeDtypeStruct(x.shape, x.dtype),
    )(x, table)


# speedup vs baseline: 1.7896x; 1.0645x over previous
import jax
import jax.numpy as jnp
from jax.experimental import pallas as pl

_TOKEN_BLOCK = 2048


def _add_kernel(x_ref, t_ref, o_ref):
    o_ref[0] = x_ref[0] + t_ref[...]


def kernel(x, table):
    batch, num_tokens, token_size = x.shape
    tb = _TOKEN_BLOCK
    grid = (num_tokens // tb, batch)
    return pl.pallas_call(
        _add_kernel,
        grid=grid,
        in_specs=[
            pl.BlockSpec((1, tb, token_size), lambda i, j: (j, i, 0)),
            pl.BlockSpec((tb, token_size), lambda i, j: (i, 0)),
        ],
        out_specs=pl.BlockSpec((1, tb, token_size), lambda i, j: (j, i, 0)),
        out_shape=jax.ShapeDtypeStruct(x.shape, x.dtype),
    )(x, table)


# speedup vs baseline: 1.8104x; 1.0116x over previous
import jax
import jax.numpy as jnp
from jax.experimental import pallas as pl

_TOKEN_BLOCK = 1024


def _add_kernel(x_ref, t_ref, o_ref):
    o_ref[...] = x_ref[...] + t_ref[...][None]


def kernel(x, table):
    batch, num_tokens, token_size = x.shape
    tb = _TOKEN_BLOCK
    grid = (num_tokens // tb,)
    return pl.pallas_call(
        _add_kernel,
        grid=grid,
        in_specs=[
            pl.BlockSpec((batch, tb, token_size), lambda i: (0, i, 0)),
            pl.BlockSpec((tb, token_size), lambda i: (i, 0)),
        ],
        out_specs=pl.BlockSpec((batch, tb, token_size), lambda i: (0, i, 0)),
        out_shape=jax.ShapeDtypeStruct(x.shape, x.dtype),
    )(x, table)
